# baseline (speedup 1.0000x reference)
import functools

import jax
import jax.numpy as jnp
from jax import lax
from jax.experimental import pallas as pl
from jax.experimental.pallas import tpu as pltpu
from jax.experimental.pallas import tpu_sc as plsc

_NC = 2
_NS = 16
_NW = _NC * _NS
_L = 16
_CHUNK = 128


def _rsqrt_newton(t):
    bits = plsc.bitcast(t, jnp.int32)
    y = plsc.bitcast(jnp.int32(0x5F3759DF) - (bits >> 1), jnp.float32)
    for _ in range(3):
        y = y * (1.5 - 0.5 * t * y * y)
    return y


def _make_kernel(B, D, V):
    assert D == 2 * _L
    b_per_w = B // _NW
    n_chunks = b_per_w // _CHUNK
    mesh = plsc.VectorSubcoreMesh(core_axis_name="c", subcore_axis_name="s")

    @functools.partial(
        pl.kernel,
        mesh=mesh,
        compiler_params=pltpu.CompilerParams(
            needs_layout_passes=False, use_tc_tiling_on_sc=False
        ),
        out_type=jax.ShapeDtypeStruct((B, D), jnp.float32),
        scratch_types=[
            pltpu.VMEM((n_chunks, _CHUNK), jnp.int32),
            pltpu.VMEM((b_per_w, D), jnp.float32),
            pltpu.VMEM((_L,), jnp.float32),
            pltpu.SemaphoreType.DMA,
        ],
    )
    def k(idx_hbm, table_hbm, out_hbm, idx_v, rows_v, scale_v, sem):
        wid = lax.axis_index("s") * _NC + lax.axis_index("c")
        base = wid * b_per_w
        pltpu.sync_copy(idx_hbm.at[wid], idx_v)
        copies = []
        for j in range(n_chunks):
            copies.append(
                pltpu.async_copy(
                    table_hbm.at[idx_v.at[j]],
                    rows_v.at[pl.ds(j * _CHUNK, _CHUNK)],
                    sem,
                )
            )
        for c in copies:
            c.wait()

        lane = lax.iota(jnp.int32, _L)

        def group_body(g, carry):
            row_ids = g * _L + lane
            acc = jnp.zeros((_L,), jnp.float32)
            for d in range(D):
                col_ids = (lane + d) & (D - 1)
                c = plsc.load_gather(rows_v, [row_ids, col_ids])
                acc = acc + c * c
            y_vec = _rsqrt_newton(acc)
            for rr in range(_L):
                r = g * _L + rr
                y = y_vec[rr]
                rows_v[r, pl.ds(0, _L)] = rows_v[r, pl.ds(0, _L)] * y
                rows_v[r, pl.ds(_L, _L)] = rows_v[r, pl.ds(_L, _L)] * y
            return carry

        lax.fori_loop(0, b_per_w // _L, group_body, 0)

        pltpu.sync_copy(rows_v, out_hbm.at[pl.ds(base, b_per_w)])

    return k


def kernel(indices, table):
    B = indices.shape[0]
    V, D = table.shape
    idx = indices.reshape(_NW, (B // _NW) // _CHUNK, _CHUNK).astype(jnp.int32)
    return _make_kernel(B, D, V)(idx, table)


# speedup vs baseline: 3.3076x; 3.3076x over previous
import functools

import jax
import jax.numpy as jnp
from jax import lax
from jax.experimental import pallas as pl
from jax.experimental.pallas import tpu as pltpu
from jax.experimental.pallas import tpu_sc as plsc

_NC = 2
_NS = 16
_NW = _NC * _NS
_L = 16


def _rsqrt_newton(t):
    bits = plsc.bitcast(t, jnp.int32)
    y = plsc.bitcast(jnp.int32(0x5F3759DF) - (bits >> 1), jnp.float32)
    for _ in range(3):
        y = y * (1.5 - 0.5 * t * y * y)
    return y


def _make_kernel(B, D, V):
    assert D == 2 * _L
    b_per_w = B // _NW
    n_groups = b_per_w // _L
    v_main = (V // 128) * 128
    mesh = plsc.VectorSubcoreMesh(core_axis_name="c", subcore_axis_name="s")

    @functools.partial(
        pl.kernel,
        mesh=mesh,
        compiler_params=pltpu.CompilerParams(needs_layout_passes=False),
        out_type=jax.ShapeDtypeStruct((B // 4, 128), jnp.float32),
        scratch_types=[
            pltpu.VMEM((b_per_w,), jnp.int32),
            pltpu.VMEM((_L, D, 128), jnp.float32),
            pltpu.VMEM((b_per_w // 4, 128), jnp.float32),
            pltpu.SemaphoreType.DMA,
        ],
    )
    def k(idx_hbm, tab_hbm, tail_hbm, out_hbm, idx_v, slabs_v, outbuf_v, sem):
        wid = lax.axis_index("s") * _NC + lax.axis_index("c")
        pltpu.sync_copy(idx_hbm.at[wid], idx_v)

        lane = lax.iota(jnp.int32, _L)

        def group_body(g, carry):
            rv = idx_v[pl.ds(g * _L, _L)]
            for rr in range(_L):
                r = rv[rr]
                main = r < v_main
                jstart = pl.multiple_of((r >> 7) * 128, 128)

                @pl.when(main)
                def _():
                    pltpu.async_copy(
                        tab_hbm.at[:, pl.ds(jstart, 128)], slabs_v.at[rr], sem
                    )

                @pl.when(jnp.logical_not(main))
                def _():
                    pltpu.async_copy(tail_hbm, slabs_v.at[rr], sem)

            for rr in range(_L):
                pltpu.make_async_copy(tail_hbm, slabs_v.at[rr], sem).wait()

            col = jnp.where(rv < v_main, rv & 127, rv - v_main)
            acc = jnp.zeros((_L,), jnp.float32)
            for d in range(D):
                c = plsc.load_gather(slabs_v, [lane, jnp.full((_L,), d, jnp.int32), col])
                acc = acc + c * c
            y = _rsqrt_newton(acc)
            row_ids = g * _L + lane
            prow = row_ids >> 2
            pbase = (row_ids & 3) * D
            for d in range(D):
                c = plsc.load_gather(slabs_v, [lane, jnp.full((_L,), d, jnp.int32), col])
                plsc.store_scatter(outbuf_v, [prow, pbase + d], c * y)
            return carry

        lax.fori_loop(0, n_groups, group_body, 0)

        pltpu.sync_copy(outbuf_v, out_hbm.at[pl.ds(wid * (b_per_w // 4), b_per_w // 4)])

    return k


def kernel(indices, table):
    B = indices.shape[0]
    V, D = table.shape
    idx = indices.reshape(_NW, B // _NW).astype(jnp.int32)
    v_main = (V // 128) * 128
    tail = jnp.pad(table[v_main:], ((0, 128 - (V - v_main)), (0, 0))).T
    out4 = _make_kernel(B, D, V)(idx, table.T, tail)
    return out4.reshape(B, D)


# speedup vs baseline: 4.1585x; 1.2572x over previous
import functools

import jax
import jax.numpy as jnp
from jax import lax
from jax.experimental import pallas as pl
from jax.experimental.pallas import tpu as pltpu
from jax.experimental.pallas import tpu_sc as plsc

_NC = 2
_NS = 16
_NW = _NC * _NS
_L = 16


def _rsqrt_newton(t):
    bits = plsc.bitcast(t, jnp.int32)
    y = plsc.bitcast(jnp.int32(0x5F3759DF) - (bits >> 1), jnp.float32)
    for _ in range(3):
        y = y * (1.5 - 0.5 * t * y * y)
    return y


def _make_kernel(B, D, V):
    assert D == 2 * _L
    b_per_w = B // _NW
    n_groups = b_per_w // _L
    v_main = (V // 128) * 128
    mesh = plsc.VectorSubcoreMesh(core_axis_name="c", subcore_axis_name="s")

    @functools.partial(
        pl.kernel,
        mesh=mesh,
        compiler_params=pltpu.CompilerParams(needs_layout_passes=False),
        out_type=jax.ShapeDtypeStruct((D, B), jnp.float32),
        scratch_types=[
            pltpu.VMEM((b_per_w,), jnp.int32),
            pltpu.VMEM((_L, D, 128), jnp.float32),
            pltpu.VMEM((_L, D), jnp.float32),
            pltpu.VMEM((D, b_per_w), jnp.float32),
            pltpu.SemaphoreType.DMA((_L,)),
        ],
    )
    def k(idx_hbm, tab_hbm, tail_hbm, out_hbm, idx_v, slabs_v, rows_v, outT_v, sems):
        wid = lax.axis_index("s") * _NC + lax.axis_index("c")
        base = wid * b_per_w
        pltpu.sync_copy(idx_hbm.at[wid], idx_v)

        lane = lax.iota(jnp.int32, _L)

        def fire(r, rr):
            main = r < v_main
            jstart = pl.multiple_of((r >> 7) * 128, 128)

            @pl.when(main)
            def _():
                pltpu.async_copy(
                    tab_hbm.at[:, pl.ds(jstart, 128)], slabs_v.at[rr], sems.at[rr]
                )

            @pl.when(jnp.logical_not(main))
            def _():
                pltpu.async_copy(tail_hbm, slabs_v.at[rr], sems.at[rr])

        def process_group(g, rv_next):
            rv = idx_v[pl.ds(g * _L, _L)]
            col = jnp.where(rv < v_main, rv & 127, rv - v_main)
            for rr in range(_L):
                pltpu.make_async_copy(tail_hbm, slabs_v.at[rr], sems.at[rr]).wait()
                cvec = jnp.full((_L,), col[rr], jnp.int32)
                rows_v[rr, pl.ds(0, _L)] = plsc.load_gather(
                    slabs_v.at[rr], [lane, cvec]
                )
                rows_v[rr, pl.ds(_L, _L)] = plsc.load_gather(
                    slabs_v.at[rr], [lane + _L, cvec]
                )
                if rv_next is not None:
                    fire(rv_next[rr], rr)
            acc = jnp.zeros((_L,), jnp.float32)
            for d in range(D):
                dcol = (lane + d) & (D - 1)
                c = plsc.load_gather(rows_v, [lane, dcol])
                acc = acc + c * c
            y = _rsqrt_newton(acc)
            row_ids = g * _L + lane
            for d in range(D):
                dcol = (lane + d) & (D - 1)
                c = plsc.load_gather(rows_v, [lane, dcol])
                plsc.store_scatter(outT_v, [dcol, row_ids], c * y)

        rv0 = idx_v[pl.ds(0, _L)]
        for rr in range(_L):
            fire(rv0[rr], rr)

        def group_body(g, carry):
            process_group(g, idx_v[pl.ds((g + 1) * _L, _L)])
            return carry

        lax.fori_loop(0, n_groups - 1, group_body, 0)
        process_group(n_groups - 1, None)

        pltpu.sync_copy(outT_v, out_hbm.at[:, pl.ds(base, b_per_w)])

    return k


def kernel(indices, table):
    B = indices.shape[0]
    V, D = table.shape
    idx = indices.reshape(_NW, B // _NW).astype(jnp.int32)
    v_main = (V // 128) * 128
    tail = jnp.pad(table[v_main:], ((0, 128 - (V - v_main)), (0, 0))).T
    out_t = _make_kernel(B, D, V)(idx, table.T, tail)
    return out_t.T
